# baseline (speedup 1.0000x reference)
import jax
import jax.numpy as jnp
from jax import lax
from jax.experimental import pallas as pl
from jax.experimental.pallas import tpu as pltpu

G = 512
D = 128
N = 100000
NB = 25
Nb = N // NB


def _fused_body(e1_ref, e2_ref, w1_ref, b1_ref, w2_ref, b2_ref, seg_ref,
                out_ref, num_ref, den_ref, m_ref):
    i = pl.program_id(0)
    e1 = e1_ref[...]
    e2 = e2_ref[...]
    wa = (w1_ref[0:D, :] + w1_ref[D:2 * D, :]).astype(jnp.bfloat16)
    wb = w1_ref[D:2 * D, :].astype(jnp.bfloat16)
    h = jnp.dot(e1.astype(jnp.bfloat16), wa, preferred_element_type=jnp.float32)
    h = h - jnp.dot(e2.astype(jnp.bfloat16), wb, preferred_element_type=jnp.float32)
    h = jnp.maximum(h + b1_ref[...], 0.0)
    raw = jnp.dot(h.astype(jnp.bfloat16), w2_ref[...].astype(jnp.bfloat16),
                  preferred_element_type=jnp.float32) + b2_ref[...]

    m_b = jnp.max(raw)
    e = jnp.exp(raw - m_b)
    seg_row = seg_ref[0]
    ids = lax.broadcasted_iota(jnp.int32, (G, Nb), 0)
    ohT = (seg_row == ids).astype(jnp.bfloat16)
    wd = ((e1 - e2) * e).astype(jnp.bfloat16)
    nb = jnp.dot(ohT, wd, preferred_element_type=jnp.float32)
    db = jnp.dot(ohT, e.astype(jnp.bfloat16),
                 preferred_element_type=jnp.float32)

    @pl.when(i == 0)
    def _init():
        m_ref[0] = m_b
        num_ref[...] = nb
        den_ref[...] = db

    @pl.when(i != 0)
    def _acc():
        m_old = m_ref[0]
        m_new = jnp.maximum(m_old, m_b)
        alpha = jnp.exp(m_old - m_new)
        beta = jnp.exp(m_b - m_new)
        num_ref[...] = num_ref[...] * alpha + nb * beta
        den_ref[...] = den_ref[...] * alpha + db * beta
        m_ref[0] = m_new

    @pl.when(i == NB - 1)
    def _final():
        den = den_ref[...]
        out_ref[...] = num_ref[...] * jnp.where(den > 0.0, 1.0 / den, 0.0)


def kernel(out_gnn, batch_input, W1, b1, W2, b2):
    seg = batch_input[:N].reshape(NB, 1, Nb)
    b1r = b1.reshape(1, D)
    b2r = b2.reshape(1, 1)

    out = pl.pallas_call(
        _fused_body,
        grid=(NB,),
        in_specs=[
            pl.BlockSpec((Nb, D), lambda i: (i, 0)),
            pl.BlockSpec((Nb, D), lambda i: (i + NB, 0)),
            pl.BlockSpec((2 * D, D), lambda i: (0, 0)),
            pl.BlockSpec((1, D), lambda i: (0, 0)),
            pl.BlockSpec((D, 1), lambda i: (0, 0)),
            pl.BlockSpec((1, 1), lambda i: (0, 0)),
            pl.BlockSpec((1, 1, Nb), lambda i: (i, 0, 0)),
        ],
        out_specs=pl.BlockSpec((G, D), lambda i: (0, 0)),
        out_shape=jax.ShapeDtypeStruct((G, D), jnp.float32),
        scratch_shapes=[
            pltpu.VMEM((G, D), jnp.float32),
            pltpu.VMEM((G, 1), jnp.float32),
            pltpu.SMEM((1,), jnp.float32),
        ],
    )(out_gnn, out_gnn, W1, b1r, W2, b2r, seg)

    return out
